# baseline (speedup 1.0000x reference)
import functools

import jax
import jax.numpy as jnp
from jax import lax
from jax.experimental import pallas as pl
from jax.experimental.pallas import tpu as pltpu
from jax.experimental.pallas import tpu_sc as plsc

DIM = 32
EMB_RANGE = 14.0 / 500.0
PI = 3.141592653589793
_PHASE_DIV = EMB_RANGE / PI

_LANES = 16
_CH = 128
_UNROLL = 8


def _rel_tables(rel_w):
    def body(rel_ref, rr_ref, ir_ref):
        ph = rel_ref[...] / jnp.float32(_PHASE_DIV)
        rr_ref[...] = jnp.cos(ph)
        ir_ref[...] = jnp.sin(ph)

    n_rel = rel_w.shape[0]
    return pl.pallas_call(
        body,
        out_shape=[jax.ShapeDtypeStruct((n_rel, DIM), jnp.float32)] * 2,
    )(rel_w)


def _vsqrt(x):
    x = jnp.maximum(x, jnp.float32(1e-30))
    i = lax.bitcast_convert_type(x, jnp.int32)
    i = jnp.int32(0x5F3759DF) - lax.shift_right_arithmetic(i, jnp.int32(1))
    y = lax.bitcast_convert_type(i, jnp.float32)
    half_x = jnp.float32(0.5) * x
    for _ in range(3):
        y = y * (jnp.float32(1.5) - half_x * y * y)
    return x * y


def _sc_score(h, t, r, re_w, im_w, rr_tab, ir_tab):
    rows = h.shape[0]
    mesh = plsc.VectorSubcoreMesh(core_axis_name="c", subcore_axis_name="s")
    nc, ns = mesh.num_cores, mesh.num_subcores
    nw = nc * ns
    bpw = rows // nw
    nch = bpw // _CH
    assert bpw * nw == rows and nch * _CH == bpw

    @functools.partial(
        pl.kernel,
        out_type=jax.ShapeDtypeStruct((rows,), jnp.float32),
        mesh=mesh,
        scratch_types=[
            pltpu.VMEM((bpw,), jnp.int32),
            pltpu.VMEM((bpw,), jnp.int32),
            pltpu.VMEM((bpw,), jnp.int32),
            pltpu.VMEM((6, _CH, DIM), jnp.float32),
            pltpu.VMEM((_LANES * _LANES,), jnp.float32),
            pltpu.VMEM((_CH,), jnp.float32),
            pltpu.SemaphoreType.DMA,
        ],
        compiler_params=pltpu.CompilerParams(
            needs_layout_passes=False, use_tc_tiling_on_sc=False),
    )
    def k(h_hbm, t_hbm, r_hbm, rew_hbm, imw_hbm, rrt_hbm, irt_hbm, out_hbm,
          hidx, tidx, ridx, buf, sc, outv, sem):
        cid = lax.axis_index("c")
        sid = lax.axis_index("s")
        wid = sid * nc + cid
        base = wid * bpw
        pltpu.sync_copy(h_hbm.at[pl.ds(base, bpw)], hidx)
        pltpu.sync_copy(t_hbm.at[pl.ds(base, bpw)], tidx)
        pltpu.sync_copy(r_hbm.at[pl.ds(base, bpw)], ridx)

        def chunk_body(cc, carry):
            off = cc * _CH
            cps = (
                pltpu.async_copy(rew_hbm.at[hidx.at[pl.ds(off, _CH)]], buf.at[0], sem),
                pltpu.async_copy(rew_hbm.at[tidx.at[pl.ds(off, _CH)]], buf.at[1], sem),
                pltpu.async_copy(imw_hbm.at[hidx.at[pl.ds(off, _CH)]], buf.at[2], sem),
                pltpu.async_copy(imw_hbm.at[tidx.at[pl.ds(off, _CH)]], buf.at[3], sem),
                pltpu.async_copy(rrt_hbm.at[ridx.at[pl.ds(off, _CH)]], buf.at[4], sem),
                pltpu.async_copy(irt_hbm.at[ridx.at[pl.ds(off, _CH)]], buf.at[5], sem),
            )
            for cp in cps:
                cp.wait()

            row_iota = lax.iota(jnp.int32, _LANES)

            def row_body(g, inner):
                for u in range(_LANES):
                    rr = g * _LANES + u
                    sv = None
                    for o in (0, _LANES):
                        rh = buf[0, rr, pl.ds(o, _LANES)]
                        rt = buf[1, rr, pl.ds(o, _LANES)]
                        ih = buf[2, rr, pl.ds(o, _LANES)]
                        it = buf[3, rr, pl.ds(o, _LANES)]
                        rrel = buf[4, rr, pl.ds(o, _LANES)]
                        irel = buf[5, rr, pl.ds(o, _LANES)]
                        re = rh * rt + irel * it - rh
                        im = rrel * it - irel * rh - ih
                        s = _vsqrt(re * re + im * im)
                        sv = s if sv is None else sv + s
                    sc[pl.ds(u * _LANES, _LANES)] = sv
                col_iota = row_iota * _LANES
                acc = None
                for i in range(_LANES):
                    col = plsc.load_gather(sc, [col_iota + i])
                    acc = col if acc is None else acc + col
                outv[pl.ds(g * _LANES, _LANES)] = jnp.float32(12.0) - acc
                return inner

            lax.fori_loop(0, _CH // _LANES, row_body, 0)
            pltpu.sync_copy(outv, out_hbm.at[pl.ds(base + off, _CH)])
            return carry

        lax.fori_loop(0, nch, chunk_body, 0)

    return k(h, t, r, re_w, im_w, rr_tab, ir_tab)


def kernel(heads, tails, relations, negative_heads, negative_tails,
           negative_relations, re_ent_w, im_ent_w, rel_w):
    b = heads.shape[0]
    rr_tab, ir_tab = _rel_tables(rel_w)
    h = jnp.concatenate([heads, negative_heads]).astype(jnp.int32)
    t = jnp.concatenate([tails, negative_tails]).astype(jnp.int32)
    r = jnp.concatenate([relations, negative_relations]).astype(jnp.int32)
    out = _sc_score(h, t, r, re_ent_w, im_ent_w, rr_tab, ir_tab)
    return out[:b], out[b:]
